# Initial kernel scaffold; baseline (speedup 1.0000x reference)
import jax
import jax.numpy as jnp
from jax.experimental import pallas as pl


def kernel(x, edge_index, W1, b1, W2, b2, fc_W, fc_b):
    raise NotImplementedError("write your pallas kernel here")


# speedup vs baseline: 15.7936x; 15.7936x over previous
import jax
import jax.numpy as jnp
from jax import lax
from jax.experimental import pallas as pl
from jax.experimental.pallas import tpu as pltpu
from jax.experimental.pallas import tpu_sc as plsc

N = 10000
E = 320000
D_FEAT = 128
HIDDEN = 64
OUT = 2

NC = 2
NS = 16
NW = NC * NS
L = 16

_MESH = dict(core_axis_name="c", subcore_axis_name="s", num_cores=NC,
             num_subcores=NS)

EPW = E // NW
CE = 8000


def _zero_pair(ref_a, ref_b, n):
    def z(i, _):
        ref_a[pl.ds(i * L, L)] = jnp.zeros((L,), jnp.float32)
        ref_b[pl.ds(i * L, L)] = jnp.zeros((L,), jnp.float32)
        return 0
    lax.fori_loop(0, n // L, z, 0)


def _deg_body(src_hbm, dst_hbm, out_hbm, src_v, dst_v, hist_o, hist_i):
    cid = lax.axis_index("c")
    sid = lax.axis_index("s")
    wid = sid * NC + cid
    base = wid * EPW
    _zero_pair(hist_o, hist_i, N)
    pltpu.sync_copy(src_hbm.at[pl.ds(base, EPW)], src_v)
    pltpu.sync_copy(dst_hbm.at[pl.ds(base, EPW)], dst_v)
    ones = jnp.ones((L,), jnp.float32)

    def step(i, _):
        s = src_v[pl.ds(i * L, L)]
        d = dst_v[pl.ds(i * L, L)]
        plsc.addupdate_scatter(hist_o, [s], ones)
        plsc.addupdate_scatter(hist_i, [d], ones)
        return 0
    lax.fori_loop(0, EPW // L, step, 0)
    pltpu.sync_copy(hist_o, out_hbm.at[wid, 0])
    pltpu.sync_copy(hist_i, out_hbm.at[wid, 1])


def _deg_call(src, dst):
    return pl.kernel(
        _deg_body,
        out_type=jax.ShapeDtypeStruct((NW, 2, N), jnp.float32),
        mesh=plsc.VectorSubcoreMesh(**_MESH),
        compiler_params=pltpu.CompilerParams(needs_layout_passes=False),
        scratch_types=[
            pltpu.VMEM((EPW,), jnp.int32),
            pltpu.VMEM((EPW,), jnp.int32),
            pltpu.VMEM((N,), jnp.float32),
            pltpu.VMEM((N,), jnp.float32),
        ],
    )(src, dst)


def _prop1_body(hsT_hbm, src_hbm, dst_hbm, out_hbm,
                src_v, dst_v, col_a, col_b, acc_a, acc_b):
    cid = lax.axis_index("c")
    sid = lax.axis_index("s")
    wid = sid * NC + cid
    r0 = wid * 2
    pltpu.sync_copy(hsT_hbm.at[r0], col_a)
    pltpu.sync_copy(hsT_hbm.at[r0 + 1], col_b)
    _zero_pair(acc_a, acc_b, N)

    def chunk(c, _):
        pltpu.sync_copy(src_hbm.at[pl.ds(c * CE, CE)], src_v)
        pltpu.sync_copy(dst_hbm.at[pl.ds(c * CE, CE)], dst_v)

        def step(i, _):
            s = src_v[pl.ds(i * L, L)]
            d = dst_v[pl.ds(i * L, L)]
            va = plsc.load_gather(col_a, [s])
            vb = plsc.load_gather(col_b, [s])
            plsc.addupdate_scatter(acc_a, [d], va)
            plsc.addupdate_scatter(acc_b, [d], vb)
            return 0
        lax.fori_loop(0, CE // L, step, 0)
        return 0
    lax.fori_loop(0, E // CE, chunk, 0)
    pltpu.sync_copy(acc_a, out_hbm.at[r0])
    pltpu.sync_copy(acc_b, out_hbm.at[r0 + 1])


def _prop1_call(hsT, src, dst):
    return pl.kernel(
        _prop1_body,
        out_type=jax.ShapeDtypeStruct((HIDDEN, N), jnp.float32),
        mesh=plsc.VectorSubcoreMesh(**_MESH),
        compiler_params=pltpu.CompilerParams(needs_layout_passes=False),
        scratch_types=[
            pltpu.VMEM((CE,), jnp.int32),
            pltpu.VMEM((CE,), jnp.int32),
            pltpu.VMEM((N,), jnp.float32),
            pltpu.VMEM((N,), jnp.float32),
            pltpu.VMEM((N,), jnp.float32),
            pltpu.VMEM((N,), jnp.float32),
        ],
    )(hsT, src, dst)


def _prop2_body(gsT_hbm, src_hbm, dst_hbm, out_hbm,
                src_v, dst_v, tab_a, tab_b, acc_a, acc_b):
    cid = lax.axis_index("c")
    sid = lax.axis_index("s")
    wid = sid * NC + cid
    base = wid * EPW
    pltpu.sync_copy(gsT_hbm.at[0], tab_a)
    pltpu.sync_copy(gsT_hbm.at[1], tab_b)
    _zero_pair(acc_a, acc_b, N)
    pltpu.sync_copy(src_hbm.at[pl.ds(base, EPW)], src_v)
    pltpu.sync_copy(dst_hbm.at[pl.ds(base, EPW)], dst_v)

    def step(i, _):
        s = src_v[pl.ds(i * L, L)]
        d = dst_v[pl.ds(i * L, L)]
        va = plsc.load_gather(tab_a, [s])
        vb = plsc.load_gather(tab_b, [s])
        plsc.addupdate_scatter(acc_a, [d], va)
        plsc.addupdate_scatter(acc_b, [d], vb)
        return 0
    lax.fori_loop(0, EPW // L, step, 0)
    pltpu.sync_copy(acc_a, out_hbm.at[wid, 0])
    pltpu.sync_copy(acc_b, out_hbm.at[wid, 1])


def _prop2_call(gsT, src, dst):
    return pl.kernel(
        _prop2_body,
        out_type=jax.ShapeDtypeStruct((NW, 2, N), jnp.float32),
        mesh=plsc.VectorSubcoreMesh(**_MESH),
        compiler_params=pltpu.CompilerParams(needs_layout_passes=False),
        scratch_types=[
            pltpu.VMEM((EPW,), jnp.int32),
            pltpu.VMEM((EPW,), jnp.int32),
            pltpu.VMEM((N,), jnp.float32),
            pltpu.VMEM((N,), jnp.float32),
            pltpu.VMEM((N,), jnp.float32),
            pltpu.VMEM((N,), jnp.float32),
        ],
    )(gsT, src, dst)


def _tcA_body(x_ref, w1_ref, degp_ref, hsT_ref, norms_ref):
    dg = jnp.sum(degp_ref[...], axis=0)
    norms = jnp.where(dg > 0, lax.rsqrt(dg), 0.0)
    norms_ref[...] = norms
    hT = lax.dot_general(w1_ref[...], x_ref[...],
                         (((0,), (1,)), ((), ())),
                         preferred_element_type=jnp.float32)
    hsT_ref[...] = hT * norms[0:1]


def _tcA_call(x, W1, deg_parts):
    return pl.pallas_call(
        _tcA_body,
        out_shape=[
            jax.ShapeDtypeStruct((HIDDEN, N), jnp.float32),
            jax.ShapeDtypeStruct((2, N), jnp.float32),
        ],
    )(x, W1, deg_parts)


def _tcB_body(aggT_ref, norms_ref, b1_ref, w2_ref, gsT_ref):
    norms = norms_ref[...]
    h1T = jnp.maximum(aggT_ref[...] * norms[1:2] + b1_ref[...], 0.0)
    gT = lax.dot_general(w2_ref[...], h1T,
                         (((0,), (0,)), ((), ())),
                         preferred_element_type=jnp.float32)
    gsT_ref[...] = gT * norms[0:1]


def _tcB_call(aggT, norms, b1c, W2):
    return pl.pallas_call(
        _tcB_body,
        out_shape=jax.ShapeDtypeStruct((OUT, N), jnp.float32),
    )(aggT, norms, b1c, W2)


def _tcC_body(parts_ref, norms_ref, b2_ref, fcw_ref, fcb_ref, out_ref):
    agg2 = jnp.sum(parts_ref[...], axis=0)
    norms = norms_ref[...]
    h2 = jnp.maximum(agg2 * norms[1:2] + b2_ref[...], 0.0)
    s = jnp.sum(h2, axis=1, keepdims=True) * (1.0 / N)
    m = jnp.max(h2, axis=1, keepdims=True)
    stats = jnp.concatenate([s, m], axis=0)
    z = jnp.sum(stats * fcw_ref[...]) + fcb_ref[0, 0]
    out_ref[...] = jnp.reshape(1.0 / (1.0 + jnp.exp(-z)), (1, 1))


def _tcC_call(parts2, norms, b2c, fc_W, fc_bc):
    return pl.pallas_call(
        _tcC_body,
        out_shape=jax.ShapeDtypeStruct((1, 1), jnp.float32),
    )(parts2, norms, b2c, fc_W, fc_bc)


def kernel(x, edge_index, W1, b1, W2, b2, fc_W, fc_b):
    src = edge_index[0]
    dst = edge_index[1]
    deg_parts = _deg_call(src, dst)
    hsT, norms = _tcA_call(x, W1, deg_parts)
    aggT = _prop1_call(hsT, src, dst)
    gsT = _tcB_call(aggT, norms, b1.reshape(HIDDEN, 1), W2)
    parts2 = _prop2_call(gsT, src, dst)
    return _tcC_call(parts2, norms, b2.reshape(OUT, 1), fc_W,
                     fc_b.reshape(1, 1))
